# baseline (speedup 1.0000x reference)
import functools

import jax
import jax.numpy as jnp
from jax import lax
from jax.experimental import pallas as pl
from jax.experimental.pallas import tpu as pltpu
from jax.experimental.pallas import tpu_sc as plsc

_MIN = -5.0
_MAX = 5.0
_EPS = 1e-06
_BINS = 1001
_D = 1536
_N = 65536

_NC = 2
_NS = 16
_L = 16
_NW = _NC * _NS
_BPW = _N // _NW
_K = 16
_NB = 4
_R = 512
_S = _BPW - _R
_NCHUNK = _S // _K
_PROWS = 1024


def _prep_body(table_ref, const_ref, out_ref):
    out_ref[...] = table_ref[...] + const_ref[...]


def _prep_table(table, const_row):
    return pl.pallas_call(
        _prep_body,
        out_shape=jax.ShapeDtypeStruct((_PROWS, _D), jnp.float32),
        grid=(_PROWS // 128,),
        in_specs=[
            pl.BlockSpec((128, _D), lambda i: (i, 0)),
            pl.BlockSpec((1, _D), lambda i: (0, 0)),
        ],
        out_specs=pl.BlockSpec((128, _D), lambda i: (i, 0)),
    )(table, const_row)


@functools.partial(
    pl.kernel,
    out_type=jax.ShapeDtypeStruct((_N, _D), jnp.float32),
    mesh=plsc.VectorSubcoreMesh(core_axis_name="c", subcore_axis_name="s"),
    scratch_types=[
        pltpu.VMEM((_PROWS,), jnp.float32),
        pltpu.VMEM((_BPW,), jnp.float32),
        pltpu.VMEM((_BPW,), jnp.int32),
        pltpu.VMEM((_K, _D), jnp.float32),
        pltpu.VMEM((_K, _D), jnp.float32),
        pltpu.VMEM((_K, _D), jnp.float32),
        pltpu.VMEM((_K, _D), jnp.float32),
        pltpu.SemaphoreType.DMA,
        pltpu.SemaphoreType.DMA,
        pltpu.SemaphoreType.DMA,
        pltpu.SemaphoreType.DMA,
        pltpu.SemaphoreType.DMA,
        pltpu.SemaphoreType.DMA,
        pltpu.SemaphoreType.DMA,
        pltpu.SemaphoreType.DMA,
        pltpu.SemaphoreType.DMA,
    ],
    compiler_params=pltpu.CompilerParams(needs_layout_passes=False),
)
def _sc_gather(x_hbm, pos_hbm, table_hbm, out_hbm, pos_v, x_v, idx_v,
               rows0, rows1, rows2, rows3, g0, g1, g2, g3, w0, w1, w2, w3, dsem):
    wid = lax.axis_index("s") * _NC + lax.axis_index("c")
    base = wid * _BPW

    pltpu.sync_copy(pos_hbm, pos_v)
    pltpu.sync_copy(x_hbm.at[pl.ds(base, _BPW)], x_v)

    lo = jnp.float32(_MIN)
    hi = jnp.float32(_MAX - _EPS)

    def dig_body(i, carry):
        off = pl.multiple_of(i * _L, _L)
        xv = x_v[pl.ds(off, _L)]
        xc = jnp.minimum(jnp.maximum(xv, lo), hi)
        f = (xc + jnp.float32(-_MIN)) * jnp.float32((_BINS - 1) / (_MAX - _MIN))
        j = (f + jnp.float32(0.5)).astype(jnp.int32)
        j = jnp.clip(j, 0, _BINS - 1)
        pj = plsc.load_gather(pos_v, [j])
        idx = j + jnp.where(pj <= xc, 1, 0).astype(jnp.int32)
        idx_v[pl.ds(off, _L)] = idx
        return carry

    lax.fori_loop(0, _BPW // _L, dig_body, 0)

    iot = lax.iota(jnp.int32, _L)

    def dma_body(g, carry):
        off = pl.multiple_of(_S + g * _L, _L)
        v16 = idx_v[pl.ds(off, _L)]
        for lane in range(_L):
            s = lax.reduce_sum(jnp.where(iot == lane, v16, 0), axes=(0,))
            pltpu.async_copy(table_hbm.at[s], out_hbm.at[base + off + lane], dsem)
        return carry

    lax.fori_loop(0, _R // _L, dma_body, 0)

    rows = (rows0, rows1, rows2, rows3)
    gsem = (g0, g1, g2, g3)
    wsem = (w0, w1, w2, w3)

    def start_gather(cb, b):
        pltpu.async_copy(table_hbm.at[idx_v.at[pl.ds(cb, _K)]], rows[b], gsem[b])

    def wait_gather(b):
        pltpu.make_async_copy(
            table_hbm.at[idx_v.at[pl.ds(0, _K)]], rows[b], gsem[b]).wait()

    def start_write(cb, b):
        pltpu.async_copy(rows[b], out_hbm.at[pl.ds(base + cb, _K)], wsem[b])

    def wait_write(b):
        pltpu.make_async_copy(rows[b], out_hbm.at[pl.ds(base, _K)], wsem[b]).wait()

    start_gather(pl.multiple_of(0, _K), 0)
    start_gather(pl.multiple_of(_K, _K), 1)

    for c in range(2):
        cb = pl.multiple_of(c * _K, _K)
        wait_gather(c)
        start_write(cb, c)
        start_gather(cb + 2 * _K, c + 2)

    def group_body(i, carry):
        for p in range(4):
            b = (2 + p) % _NB
            cb = pl.multiple_of((2 + 4 * i + p) * _K, _K)
            wait_gather(b)
            start_write(cb, b)
            wait_write((b + 2) % _NB)
            start_gather(cb + 2 * _K, (b + 2) % _NB)
        return carry

    lax.fori_loop(0, (_NCHUNK - 4) // 4, group_body, 0)

    for p in range(2):
        c = _NCHUNK - 2 + p
        b = c % _NB
        cb = pl.multiple_of(c * _K, _K)
        wait_gather(b)
        start_write(cb, b)
    for b in range(_NB):
        wait_write(b)
    pltpu.make_async_copy(table_hbm.at[pl.ds(0, _R)],
                          out_hbm.at[pl.ds(base + _S, _R)], dsem).wait()


def kernel(x, pos_embeddings, learnable_constant):
    positions = jnp.linspace(_MIN, _MAX, _BINS).astype(jnp.float32)
    pos_pad = jnp.zeros((_PROWS,), jnp.float32).at[:_BINS].set(positions)
    table_plus = _prep_table(pos_embeddings, learnable_constant.reshape(1, _D))
    return _sc_gather(x, pos_pad, table_plus)


# speedup vs baseline: 9.9922x; 9.9922x over previous
import functools

import jax
import jax.numpy as jnp
from jax import lax
from jax.experimental import pallas as pl
from jax.experimental.pallas import tpu as pltpu
from jax.experimental.pallas import tpu_sc as plsc

_MIN = -5.0
_MAX = 5.0
_EPS = 1e-06
_BINS = 1001
_D = 1536
_N = 65536

_NC = 2
_NS = 16
_L = 16
_NW = _NC * _NS
_BPW = _N // _NW
_K = 32
_NCHUNK = _BPW // _K
_PROWS = 1024


def _prep_body(table_ref, const_ref, out_ref):
    out_ref[...] = table_ref[...] + const_ref[...]


def _prep_table(table, const_row):
    return pl.pallas_call(
        _prep_body,
        out_shape=jax.ShapeDtypeStruct((_PROWS, _D), jnp.float32),
        grid=(_PROWS // 128,),
        in_specs=[
            pl.BlockSpec((128, _D), lambda i: (i, 0)),
            pl.BlockSpec((1, _D), lambda i: (0, 0)),
        ],
        out_specs=pl.BlockSpec((128, _D), lambda i: (i, 0)),
    )(table, const_row)


@functools.partial(
    pl.kernel,
    out_type=jax.ShapeDtypeStruct((_N, _D), jnp.float32),
    mesh=plsc.VectorSubcoreMesh(core_axis_name="c", subcore_axis_name="s"),
    scratch_types=[
        pltpu.VMEM((_PROWS,), jnp.float32),
        pltpu.VMEM((_BPW,), jnp.float32),
        pltpu.VMEM((_BPW,), jnp.int32),
        pltpu.VMEM((_K, _D), jnp.float32),
        pltpu.VMEM((_K, _D), jnp.float32),
        pltpu.SemaphoreType.DMA,
        pltpu.SemaphoreType.DMA,
        pltpu.SemaphoreType.DMA,
        pltpu.SemaphoreType.DMA,
    ],
    compiler_params=pltpu.CompilerParams(needs_layout_passes=False),
)
def _sc_gather(x_hbm, pos_hbm, table_hbm, out_hbm, pos_v, x_v, idx_v,
               rows0, rows1, g0, g1, w0, w1):
    wid = lax.axis_index("s") * _NC + lax.axis_index("c")
    base = wid * _BPW

    pltpu.sync_copy(pos_hbm, pos_v)
    pltpu.sync_copy(x_hbm.at[pl.ds(base, _BPW)], x_v)

    lo = jnp.float32(_MIN)
    hi = jnp.float32(_MAX - _EPS)

    def dig_body(i, carry):
        off = pl.multiple_of(i * _L, _L)
        xv = x_v[pl.ds(off, _L)]
        xc = jnp.minimum(jnp.maximum(xv, lo), hi)
        f = (xc + jnp.float32(-_MIN)) * jnp.float32((_BINS - 1) / (_MAX - _MIN))
        j = (f + jnp.float32(0.5)).astype(jnp.int32)
        j = jnp.clip(j, 0, _BINS - 1)
        pj = plsc.load_gather(pos_v, [j])
        idx = j + jnp.where(pj <= xc, 1, 0).astype(jnp.int32)
        idx_v[pl.ds(off, _L)] = idx
        return carry

    lax.fori_loop(0, (2 * _K) // _L, dig_body, 0)

    rows = (rows0, rows1)
    gsem = (g0, g1)
    wsem = (w0, w1)

    def start_gather(cb, b):
        pltpu.async_copy(table_hbm.at[idx_v.at[pl.ds(cb, _K)]], rows[b], gsem[b])

    def wait_gather(b):
        pltpu.make_async_copy(
            table_hbm.at[idx_v.at[pl.ds(0, _K)]], rows[b], gsem[b]).wait()

    def start_write(cb, b):
        pltpu.async_copy(rows[b], out_hbm.at[pl.ds(base + cb, _K)], wsem[b])

    def wait_write(b):
        pltpu.make_async_copy(rows[b], out_hbm.at[pl.ds(base, _K)], wsem[b]).wait()

    start_gather(pl.multiple_of(0, _K), 0)
    start_gather(pl.multiple_of(_K, _K), 1)

    lax.fori_loop((2 * _K) // _L, _BPW // _L, dig_body, 0)

    def pair_body(i, carry):
        for b in range(2):
            cb = pl.multiple_of((2 * i + b) * _K, _K)
            wait_gather(b)
            start_write(cb, b)
            wait_write(b)
            start_gather(cb + 2 * _K, b)
        return carry

    lax.fori_loop(0, (_NCHUNK - 2) // 2, pair_body, 0)

    for b in range(2):
        cb = pl.multiple_of((_NCHUNK - 2 + b) * _K, _K)
        wait_gather(b)
        start_write(cb, b)
    wait_write(0)
    wait_write(1)


def kernel(x, pos_embeddings, learnable_constant):
    positions = jnp.linspace(_MIN, _MAX, _BINS).astype(jnp.float32)
    pos_pad = jnp.zeros((_PROWS,), jnp.float32).at[:_BINS].set(positions)
    table_plus = _prep_table(pos_embeddings, learnable_constant.reshape(1, _D))
    return _sc_gather(x, pos_pad, table_plus)
